# baseline (speedup 1.0000x reference)
import jax
import jax.numpy as jnp
from jax.experimental import pallas as pl
from jax.experimental.pallas import tpu as pltpu

N_BINS = 10
ROWS = 100000
COLS = 1000
BR = 800
NCHUNK = ROWS // BR
NBUF = 8
C2E = 2 * 1.4426950408889634


def _ece_body(logits_ref, labels_ref, bounds_ref, bins_ref, ece_ref,
              buf, sems):
    c = pl.program_id(0)

    def start(i):
        pltpu.make_async_copy(
            logits_ref.at[pl.ds(pl.multiple_of(i * BR, 8), BR), :],
            buf.at[jax.lax.rem(i, NBUF)],
            sems.at[jax.lax.rem(i, NBUF)],
        ).start()

    def wait(i):
        pltpu.make_async_copy(
            logits_ref.at[pl.ds(pl.multiple_of(i * BR, 8), BR), :],
            buf.at[jax.lax.rem(i, NBUF)],
            sems.at[jax.lax.rem(i, NBUF)],
        ).wait()

    @pl.when(c == 0)
    def _prologue():
        bins_ref[...] = jnp.zeros_like(bins_ref)
        for i in range(NBUF):
            start(i)

    wait(c)
    x = buf[jax.lax.rem(c, NBUF)]
    e = jnp.exp2(x * C2E)
    me = jnp.max(e, axis=1, keepdims=True)
    s = jax.lax.dot_general(
        e, jnp.ones((COLS, 8), jnp.float32), (((1,), (0,)), ((), ())),
        preferred_element_type=jnp.float32)[:, 0:1]

    @pl.when(c + NBUF < NCHUNK)
    def _():
        start(c + NBUF)

    lab = labels_ref[...]
    lane = jax.lax.broadcasted_iota(jnp.int32, (1, COLS), 1)
    el = jnp.max(jnp.where(lane == lab, e, -jnp.inf),
                 axis=1, keepdims=True)
    correct = (el == me).astype(jnp.float32)
    conf = me / s

    lower = bounds_ref[0:1, 0:N_BINS]
    upper = bounds_ref[0:1, 1:N_BINS + 1]
    in_bin = ((conf > lower) & (conf <= upper)).astype(jnp.float32)
    bins_ref[0:1, :] += jnp.sum(in_bin, axis=0, keepdims=True)
    bins_ref[1:2, :] += jnp.sum(correct * in_bin, axis=0, keepdims=True)
    bins_ref[2:3, :] += jnp.sum(conf * in_bin, axis=0, keepdims=True)

    @pl.when(c == NCHUNK - 1)
    def _finish():
        cnt = bins_ref[0:1, :]
        acc = bins_ref[1:2, :]
        cfs = bins_ref[2:3, :]
        safe = jnp.maximum(cnt, 1.0)
        contrib = jnp.where(
            cnt > 0.0,
            jnp.abs(cfs / safe - acc / safe) * (cnt / float(ROWS)),
            0.0,
        )
        ece_ref[...] = jnp.sum(contrib, keepdims=True).reshape(1, 1)


def kernel(logits, labels):
    labels2d = labels.astype(jnp.int32).reshape(ROWS, 1)
    bounds = jnp.linspace(0.0, 1.0, N_BINS + 1).reshape(1, N_BINS + 1)
    bins, ece = pl.pallas_call(
        _ece_body,
        grid=(NCHUNK,),
        in_specs=[
            pl.BlockSpec(memory_space=pl.ANY),
            pl.BlockSpec((BR, 1), lambda i: (i, 0)),
            pl.BlockSpec((1, N_BINS + 1), lambda i: (0, 0)),
        ],
        out_specs=[
            pl.BlockSpec((3, N_BINS), lambda i: (0, 0)),
            pl.BlockSpec((1, 1), lambda i: (0, 0)),
        ],
        out_shape=[
            jax.ShapeDtypeStruct((3, N_BINS), jnp.float32),
            jax.ShapeDtypeStruct((1, 1), jnp.float32),
        ],
        scratch_shapes=[
            pltpu.VMEM((NBUF, BR, COLS), jnp.float32),
            pltpu.SemaphoreType.DMA((NBUF,)),
        ],
    )(logits, labels2d, bounds)
    return ece.reshape(1)


# speedup vs baseline: 1.0802x; 1.0802x over previous
import jax
import jax.numpy as jnp
from jax.experimental import pallas as pl
from jax.experimental.pallas import tpu as pltpu

N_BINS = 10
ROWS = 100000
COLS = 1000
BR = 2000
NCHUNK = ROWS // BR
NBUF = 3
C2E = 2 * 1.4426950408889634


def _ece_body(logits_ref, labels_ref, bounds_ref, bins_ref, ece_ref,
              buf, sems):
    c = pl.program_id(0)

    def start(i):
        pltpu.make_async_copy(
            logits_ref.at[pl.ds(pl.multiple_of(i * BR, 8), BR), :],
            buf.at[jax.lax.rem(i, NBUF)],
            sems.at[jax.lax.rem(i, NBUF)],
        ).start()

    def wait(i):
        pltpu.make_async_copy(
            logits_ref.at[pl.ds(pl.multiple_of(i * BR, 8), BR), :],
            buf.at[jax.lax.rem(i, NBUF)],
            sems.at[jax.lax.rem(i, NBUF)],
        ).wait()

    @pl.when(c == 0)
    def _prologue():
        bins_ref[...] = jnp.zeros_like(bins_ref)
        for i in range(NBUF):
            start(i)

    wait(c)
    x = buf[jax.lax.rem(c, NBUF)]
    e = jnp.exp2(x * C2E)
    me = jnp.max(e, axis=1, keepdims=True)
    s = jax.lax.dot_general(
        e, jnp.ones((COLS, 8), jnp.float32), (((1,), (0,)), ((), ())),
        preferred_element_type=jnp.float32)[:, 0:1]

    @pl.when(c + NBUF < NCHUNK)
    def _():
        start(c + NBUF)

    lab = labels_ref[...]
    lane = jax.lax.broadcasted_iota(jnp.int32, (1, COLS), 1)
    el = jnp.max(jnp.where(lane == lab, e, -jnp.inf),
                 axis=1, keepdims=True)
    correct = (el == me).astype(jnp.float32)
    conf = me / s

    lower = bounds_ref[0:1, 0:N_BINS]
    upper = bounds_ref[0:1, 1:N_BINS + 1]
    in_bin = ((conf > lower) & (conf <= upper)).astype(jnp.float32)
    bins_ref[0:1, :] += jnp.sum(in_bin, axis=0, keepdims=True)
    bins_ref[1:2, :] += jnp.sum(correct * in_bin, axis=0, keepdims=True)
    bins_ref[2:3, :] += jnp.sum(conf * in_bin, axis=0, keepdims=True)

    @pl.when(c == NCHUNK - 1)
    def _finish():
        cnt = bins_ref[0:1, :]
        acc = bins_ref[1:2, :]
        cfs = bins_ref[2:3, :]
        safe = jnp.maximum(cnt, 1.0)
        contrib = jnp.where(
            cnt > 0.0,
            jnp.abs(cfs / safe - acc / safe) * (cnt / float(ROWS)),
            0.0,
        )
        ece_ref[...] = jnp.sum(contrib, keepdims=True).reshape(1, 1)


def kernel(logits, labels):
    labels2d = labels.astype(jnp.int32).reshape(ROWS, 1)
    bounds = jnp.linspace(0.0, 1.0, N_BINS + 1).reshape(1, N_BINS + 1)
    bins, ece = pl.pallas_call(
        _ece_body,
        grid=(NCHUNK,),
        in_specs=[
            pl.BlockSpec(memory_space=pl.ANY),
            pl.BlockSpec((BR, 1), lambda i: (i, 0)),
            pl.BlockSpec((1, N_BINS + 1), lambda i: (0, 0)),
        ],
        out_specs=[
            pl.BlockSpec((3, N_BINS), lambda i: (0, 0)),
            pl.BlockSpec((1, 1), lambda i: (0, 0)),
        ],
        out_shape=[
            jax.ShapeDtypeStruct((3, N_BINS), jnp.float32),
            jax.ShapeDtypeStruct((1, 1), jnp.float32),
        ],
        scratch_shapes=[
            pltpu.VMEM((NBUF, BR, COLS), jnp.float32),
            pltpu.SemaphoreType.DMA((NBUF,)),
        ],
    )(logits, labels2d, bounds)
    return ece.reshape(1)
